# baseline (speedup 1.0000x reference)
import functools

import jax
import jax.numpy as jnp
from jax import lax
from jax.experimental import pallas as pl
from jax.experimental.pallas import tpu as pltpu
from jax.experimental.pallas import tpu_sc as plsc

NC = 2
NS = 16
L = 16
NW = NC * NS

BATCH = 16384
D = 64
NV = 1000000
BPW = BATCH // NW
NCOL = 128
NSLAB = (NV + NCOL - 1) // NCOL
SPW = (NSLAB + NW - 1) // NW
NCH = 4
NRING = 8
NOUT = 8
CAP = BATCH


def _make_mesh():
    return plsc.VectorSubcoreMesh(
        core_axis_name="c", subcore_axis_name="s",
        num_cores=NC, num_subcores=NS)


_params = pltpu.CompilerParams(
    needs_layout_passes=False, use_tc_tiling_on_sc=True,
    disable_bounds_checks=True)


@functools.cache
def _build_sweep():
  return functools.partial(
    pl.kernel,
    out_type=(jax.ShapeDtypeStruct((BATCH * D,), jnp.float32),
              jax.ShapeDtypeStruct((BATCH * D,), jnp.float32)),
    mesh=_make_mesh(),
    compiler_params=_params,
    scratch_types=[
        pltpu.VMEM((BATCH,), jnp.int32),
        pltpu.VMEM((CAP + L,), jnp.int32),
        pltpu.VMEM((CAP + L,), jnp.int32),
        pltpu.VMEM((2 * L,), jnp.int32),
        pltpu.VMEM((2 * L,), jnp.int32),
        pltpu.VMEM((NRING, D, NCOL), jnp.float32),
        pltpu.VMEM((NOUT, D), jnp.float32),
        pltpu.SemaphoreType.DMA,
        pltpu.SemaphoreType.DMA,
        pltpu.SemaphoreType.DMA,
        pltpu.SemaphoreType.DMA,
        pltpu.SemaphoreType.DMA,
    ],
)(_sweep_body)


def _sweep_body(utab_hbm, itab_hbm, uidx_hbm, iidx_hbm,
                uvecs_hbm, ivecs_hbm,
                stage_v, lb_v, li_v, mb_v, mi_v, ring_v, colring_v,
                semO, semA, semB, semC, semD):
    wid = lax.axis_index("s") * NC + lax.axis_index("c")
    col0 = wid * SPW
    nsl = jnp.minimum(SPW, NSLAB - col0)
    sems = (semA, semB, semC, semD)
    iota = lax.iota(jnp.int32, L)

    def sread(ref, j):
        return ref[pl.ds(j, L)][0]

    def one_table(tab_hbm, idx_hbm, vecs_hbm):
        pltpu.sync_copy(idx_hbm, stage_v)

        def comp(q, cnt):
            iv = stage_v[pl.ds(q * L, L)]
            s = iv >> 7
            m = (s >= col0) & (s < col0 + nsl)
            plsc.store_compressed(lb_v.at[pl.ds(cnt, L)], iota + q * L, mask=m)
            plsc.store_compressed(li_v.at[pl.ds(cnt, L)], iv, mask=m)
            return cnt + plsc.all_reduce_population_count(m)[0]

        cnt = lax.fori_loop(0, BATCH // L, comp, jnp.int32(0))
        nchunk = (cnt + L - 1) >> 4

        def fetch(d, sem):
            off = pl.multiple_of((col0 + d) * NCOL, NCOL)
            return pltpu.async_copy(
                tab_hbm.at[:, pl.ds(off, NCOL)], ring_v.at[d % NRING], sem)

        def wait_slab(d, sem):
            pltpu.make_async_copy(
                tab_hbm.at[:, pl.ds(0, NCOL)], ring_v.at[d % NRING], sem
            ).wait()

        def wait_out():
            pltpu.make_async_copy(
                vecs_hbm.at[pl.ds(0, D)], colring_v.at[0], semO).wait()

        def drain(n):
            lax.fori_loop(0, n, lambda _, c: (wait_out(), c)[1], 0)

        def extract(d, nout):
            col = col0 + d
            slot = d % NRING
            slotv = jnp.full((L,), slot, jnp.int32)

            def chunk(k, nout):
                bv = lb_v[pl.ds(k * L, L)]
                ivv = li_v[pl.ds(k * L, L)]
                m = ((ivv >> 7) == col) & (iota < (cnt - k * L))
                plsc.store_compressed(mb_v.at[pl.ds(0, L)], bv, mask=m)
                plsc.store_compressed(mi_v.at[pl.ds(0, L)], ivv, mask=m)
                nm = plsc.all_reduce_population_count(m)[0]

                def emit(t, nout):
                    @pl.when((nout > 0) & ((nout & (NOUT - 1)) == 0))
                    def _():
                        drain(NOUT)

                    b = sread(mb_v, t)
                    lane = jnp.full((L,), sread(mi_v, t) & (NCOL - 1),
                                    jnp.int32)
                    cslot = nout & (NOUT - 1)
                    for k2 in range(D // L):
                        v = plsc.load_gather(
                            ring_v, [slotv, iota + k2 * L, lane])
                        colring_v[cslot, pl.ds(k2 * L, L)] = v
                    pltpu.async_copy(
                        colring_v.at[cslot], vecs_hbm.at[pl.ds(b * D, D)],
                        semO)
                    return nout + 1

                return lax.fori_loop(0, nm, emit, nout)

            return lax.fori_loop(0, nchunk, chunk, nout)

        fetch(0, sems[0])
        for k in range(1, NCH):
            @pl.when(k < nsl)
            def _(k=k):
                fetch(k, sems[k])

        def slab_step(d, k, nout):
            def go(nout):
                wait_slab(d, sems[k])
                nout2 = extract(d, nout)

                @pl.when(d + NCH < nsl)
                def _():
                    fetch(d + NCH, sems[k])

                return nout2

            return lax.cond(d < nsl, go, lambda n: n, nout)

        def loop_body(p, nout):
            for k in range(NCH):
                nout = slab_step(p * NCH + k, k, nout)
            return nout

        nout = lax.fori_loop(0, (SPW + NCH - 1) // NCH, loop_body,
                             jnp.int32(0))

        rem = nout & (NOUT - 1)
        drain(jnp.where((nout > 0) & (rem == 0), NOUT, rem))

    one_table(utab_hbm, uidx_hbm, uvecs_hbm)
    one_table(itab_hbm, iidx_hbm, ivecs_hbm)


@functools.cache
def _build_dot_kernel():
  return functools.partial(
    pl.kernel,
    out_type=jax.ShapeDtypeStruct((BATCH,), jnp.float32),
    mesh=_make_mesh(),
    compiler_params=_params,
    scratch_types=[
        pltpu.VMEM((BPW * D,), jnp.float32),
        pltpu.VMEM((BPW * D,), jnp.float32),
        pltpu.VMEM((BPW,), jnp.float32),
    ],
)(_dot_body)


def _dot_body(uvecs_hbm, ivecs_hbm, out_hbm, ru_v, ri_v, out_v):
    wid = lax.axis_index("s") * NC + lax.axis_index("c")
    base = wid * BPW
    iota = lax.iota(jnp.int32, L)

    pltpu.sync_copy(uvecs_hbm.at[pl.ds(base * D, BPW * D)], ru_v)
    pltpu.sync_copy(ivecs_hbm.at[pl.ds(base * D, BPW * D)], ri_v)

    def block(rb, carry):
        rowbase = rb * (L * D) + iota * D
        accs = [jnp.zeros((L,), jnp.float32) for _ in range(4)]
        for c0 in range(D):
            flat = rowbase + ((iota + c0) & (D - 1))
            u = plsc.load_gather(ru_v, [flat])
            v = plsc.load_gather(ri_v, [flat])
            accs[c0 % 4] = accs[c0 % 4] + u * v
        out_v[pl.ds(rb * L, L)] = (accs[0] + accs[1]) + (accs[2] + accs[3])
        return carry

    lax.fori_loop(0, BPW // L, block, 0)
    pltpu.sync_copy(out_v, out_hbm.at[pl.ds(base, BPW)])


def kernel(user_indices, item_indices, user_table, item_table):
    uvecs, ivecs = _build_sweep()(
        user_table.T, item_table.T,
        user_indices.astype(jnp.int32), item_indices.astype(jnp.int32))
    return _build_dot_kernel()(uvecs, ivecs)


# speedup vs baseline: 1.4889x; 1.4889x over previous
import functools

import jax
import jax.numpy as jnp
from jax import lax
from jax.experimental import pallas as pl
from jax.experimental.pallas import tpu as pltpu
from jax.experimental.pallas import tpu_sc as plsc

NC = 2
NS = 16
L = 16
NW = NC * NS

BATCH = 16384
D = 64
NV = 1000000
BPW = BATCH // NW
NCOL = 128
NSLAB = (NV + NCOL - 1) // NCOL
SPW = (NSLAB + NW - 1) // NW
NCH = 4
NRING = 6
NOUT = 8
CAP = BATCH


def _make_mesh():
    return plsc.VectorSubcoreMesh(
        core_axis_name="c", subcore_axis_name="s",
        num_cores=NC, num_subcores=NS)


_params = pltpu.CompilerParams(
    needs_layout_passes=False, use_tc_tiling_on_sc=True,
    disable_bounds_checks=True)


@functools.cache
def _build_sweep():
  return functools.partial(
    pl.kernel,
    out_type=(jax.ShapeDtypeStruct((BATCH * D,), jnp.float32),
              jax.ShapeDtypeStruct((BATCH * D,), jnp.float32)),
    mesh=_make_mesh(),
    compiler_params=_params,
    scratch_types=[
        pltpu.VMEM((BATCH,), jnp.int32),
        pltpu.VMEM((CAP + L,), jnp.int32),
        pltpu.VMEM((CAP + L,), jnp.int32),
        pltpu.VMEM((SPW + 2 * L,), jnp.int32),
        pltpu.VMEM((SPW + 2 * L,), jnp.int32),
        pltpu.VMEM((SPW + 2 * L,), jnp.int32),
        pltpu.VMEM((NRING, D, NCOL), jnp.float32),
        pltpu.VMEM((NOUT, D), jnp.float32),
        pltpu.SemaphoreType.DMA,
        pltpu.SemaphoreType.DMA,
        pltpu.SemaphoreType.DMA,
        pltpu.SemaphoreType.DMA,
        pltpu.SemaphoreType.DMA,
    ],
)(_sweep_body)


def _sweep_body(utab_hbm, itab_hbm, uidx_hbm, iidx_hbm,
                uvecs_hbm, ivecs_hbm,
                stage_v, lb_v, lbB_v, cnts_v, curs_v, offs_v,
                ring_v, colring_v,
                semO, semA, semB, semC, semD):
    wid = lax.axis_index("s") * NC + lax.axis_index("c")
    col0 = wid * SPW
    nsl = jnp.minimum(SPW, NSLAB - col0)
    sems = (semA, semB, semC, semD)
    iota = lax.iota(jnp.int32, L)
    lane0 = iota == 0

    def sread(ref, j):
        return ref[pl.ds(j, L)][0]

    def swrite(ref, j, val):
        plsc.store_scatter(ref, [jnp.full((L,), j, jnp.int32)],
                           jnp.full((L,), val, jnp.int32), mask=lane0)

    def one_table(tab_hbm, idx_hbm, vecs_hbm):
        pltpu.sync_copy(idx_hbm, stage_v)

        def comp(q, cnt):
            iv = stage_v[pl.ds(q * L, L)]
            s = iv >> 7
            m = (s >= col0) & (s < col0 + nsl)
            plsc.store_compressed(lb_v.at[pl.ds(cnt, L)], iota + q * L, mask=m)
            return cnt + plsc.all_reduce_population_count(m)[0]

        cnt = lax.fori_loop(0, BATCH // L, comp, jnp.int32(0))

        zero = jnp.zeros((L,), jnp.int32)
        for z in range((SPW + 2 * L - 1) // L):
            cnts_v[pl.ds(z * L, L)] = zero

        def count(j, carry):
            d = (sread(stage_v, sread(lb_v, j)) >> 7) - col0
            swrite(cnts_v, d, sread(cnts_v, d) + 1)
            return carry

        lax.fori_loop(0, cnt, count, 0)

        def prefix(d, acc):
            swrite(curs_v, d, acc)
            swrite(offs_v, d, acc)
            return acc + sread(cnts_v, d)

        lax.fori_loop(0, nsl, prefix, jnp.int32(0))

        def place(j, carry):
            b = sread(lb_v, j)
            d = (sread(stage_v, b) >> 7) - col0
            o = sread(curs_v, d)
            swrite(lbB_v, o, b)
            swrite(curs_v, d, o + 1)
            return carry

        lax.fori_loop(0, cnt, place, 0)

        def fetch(d, sem):
            off = pl.multiple_of((col0 + d) * NCOL, NCOL)
            return pltpu.async_copy(
                tab_hbm.at[:, pl.ds(off, NCOL)], ring_v.at[d % NRING], sem)

        def wait_slab(d, sem):
            pltpu.make_async_copy(
                tab_hbm.at[:, pl.ds(0, NCOL)], ring_v.at[d % NRING], sem
            ).wait()

        def wait_out():
            pltpu.make_async_copy(
                vecs_hbm.at[pl.ds(0, D)], colring_v.at[0], semO).wait()

        def drain(n):
            lax.fori_loop(0, n, lambda _, c: (wait_out(), c)[1], 0)

        def extract(d, nout):
            slot = d % NRING
            slotv = jnp.full((L,), slot, jnp.int32)
            j0 = sread(offs_v, d)

            def emit(j, nout):
                @pl.when((nout > 0) & ((nout & (NOUT - 1)) == 0))
                def _():
                    drain(NOUT)

                b = sread(lbB_v, j)
                lane = jnp.full((L,), sread(stage_v, b) & (NCOL - 1),
                                jnp.int32)
                cslot = nout & (NOUT - 1)
                for k2 in range(D // L):
                    v = plsc.load_gather(
                        ring_v, [slotv, iota + k2 * L, lane])
                    colring_v[cslot, pl.ds(k2 * L, L)] = v
                pltpu.async_copy(
                    colring_v.at[cslot], vecs_hbm.at[pl.ds(b * D, D)],
                    semO)
                return nout + 1

            return lax.fori_loop(j0, j0 + sread(cnts_v, d), emit, nout)

        fetch(0, sems[0])
        for k in range(1, NCH):
            @pl.when(k < nsl)
            def _(k=k):
                fetch(k, sems[k])

        def slab_step(d, k, nout):
            def go(nout):
                wait_slab(d, sems[k])
                nout2 = extract(d, nout)

                @pl.when(d + NCH < nsl)
                def _():
                    fetch(d + NCH, sems[k])

                return nout2

            return lax.cond(d < nsl, go, lambda n: n, nout)

        def loop_body(p, nout):
            for k in range(NCH):
                nout = slab_step(p * NCH + k, k, nout)
            return nout

        nout = lax.fori_loop(0, (SPW + NCH - 1) // NCH, loop_body,
                             jnp.int32(0))

        rem = nout & (NOUT - 1)
        drain(jnp.where((nout > 0) & (rem == 0), NOUT, rem))

    one_table(utab_hbm, uidx_hbm, uvecs_hbm)
    one_table(itab_hbm, iidx_hbm, ivecs_hbm)


@functools.cache
def _build_dot_kernel():
  return functools.partial(
    pl.kernel,
    out_type=jax.ShapeDtypeStruct((BATCH,), jnp.float32),
    mesh=_make_mesh(),
    compiler_params=_params,
    scratch_types=[
        pltpu.VMEM((BPW * D,), jnp.float32),
        pltpu.VMEM((BPW * D,), jnp.float32),
        pltpu.VMEM((BPW,), jnp.float32),
    ],
)(_dot_body)


def _dot_body(uvecs_hbm, ivecs_hbm, out_hbm, ru_v, ri_v, out_v):
    wid = lax.axis_index("s") * NC + lax.axis_index("c")
    base = wid * BPW
    iota = lax.iota(jnp.int32, L)

    pltpu.sync_copy(uvecs_hbm.at[pl.ds(base * D, BPW * D)], ru_v)
    pltpu.sync_copy(ivecs_hbm.at[pl.ds(base * D, BPW * D)], ri_v)

    def block(rb, carry):
        rowbase = rb * (L * D) + iota * D
        accs = [jnp.zeros((L,), jnp.float32) for _ in range(4)]
        for c0 in range(D):
            flat = rowbase + ((iota + c0) & (D - 1))
            u = plsc.load_gather(ru_v, [flat])
            v = plsc.load_gather(ri_v, [flat])
            accs[c0 % 4] = accs[c0 % 4] + u * v
        out_v[pl.ds(rb * L, L)] = (accs[0] + accs[1]) + (accs[2] + accs[3])
        return carry

    lax.fori_loop(0, BPW // L, block, 0)
    pltpu.sync_copy(out_v, out_hbm.at[pl.ds(base, BPW)])


def kernel(user_indices, item_indices, user_table, item_table):
    uvecs, ivecs = _build_sweep()(
        user_table.T, item_table.T,
        user_indices.astype(jnp.int32), item_indices.astype(jnp.int32))
    return _build_dot_kernel()(uvecs, ivecs)
